# baseline (speedup 1.0000x reference)
import functools

import jax
import jax.numpy as jnp
from jax import lax
from jax.experimental import pallas as pl
from jax.experimental.pallas import tpu as pltpu
from jax.experimental.pallas import tpu_sc as plsc

_B = 16384
_N = 26
_D = 128
_VOCAB_PER_FIELD = 1000
_LOG2_B = 14
_BN = _B * _N
_NW = 32
_BPW = _BN // _NW
_C = 128
_NCHUNK = _BPW // _C
_NBUF = 4

_mesh = plsc.VectorSubcoreMesh(core_axis_name="c", subcore_axis_name="s")


@functools.partial(
    pl.kernel,
    mesh=_mesh,
    out_type=jax.ShapeDtypeStruct((_BN, _D), jnp.float32),
    scratch_types=[
        pltpu.VMEM((_BPW,), jnp.int32),
        pltpu.VMEM((_BPW,), jnp.int32),
        pltpu.VMEM((_C, _D), jnp.float32),
        pltpu.VMEM((_C, _D), jnp.float32),
        pltpu.VMEM((_C, _D), jnp.float32),
        pltpu.VMEM((_C, _D), jnp.float32),
        pltpu.SemaphoreType.DMA,
        pltpu.SemaphoreType.DMA,
        pltpu.SemaphoreType.DMA,
        pltpu.SemaphoreType.DMA,
        pltpu.SemaphoreType.DMA,
    ],
)
def _emb_lookup(x_hbm, table_hbm, out_hbm,
                xv, idx_v, rows0, rows1, rows2, rows3,
                gsem, osem0, osem1, osem2, osem3):
    wid = lax.axis_index("s") * 2 + lax.axis_index("c")
    base = wid * _BPW

    pltpu.sync_copy(x_hbm.at[pl.ds(base, _BPW)], xv)

    def compute_idx(c):
        def add_body(i, _):
            sl = pl.ds(c * _C + i * 16, 16)
            pos = lax.iota(jnp.int32, 16) + (base + c * _C + i * 16)
            off = lax.shift_right_logical(pos, _LOG2_B) * _VOCAB_PER_FIELD
            idx_v[sl] = xv[sl] + off
            return ()

        lax.fori_loop(0, _C // 16, add_body, ())

    def chunk(c, rows, osem):
        @pl.when(c >= _NBUF)
        def _():
            pltpu.make_async_copy(
                rows, out_hbm.at[pl.ds(base + (c - _NBUF) * _C, _C)], osem
            ).wait()

        cp = pltpu.async_copy(
            table_hbm.at[idx_v.at[pl.ds(c * _C, _C)]], rows, gsem
        )

        @pl.when(c + 1 < _NCHUNK)
        def _():
            compute_idx(c + 1)

        cp.wait()
        pltpu.async_copy(rows, out_hbm.at[pl.ds(base + c * _C, _C)], osem)

    compute_idx(0)

    def body(i, _):
        chunk(_NBUF * i, rows0, osem0)
        chunk(_NBUF * i + 1, rows1, osem1)
        chunk(_NBUF * i + 2, rows2, osem2)
        chunk(_NBUF * i + 3, rows3, osem3)
        return ()

    lax.fori_loop(0, _NCHUNK // _NBUF, body, ())

    pltpu.make_async_copy(
        rows0, out_hbm.at[pl.ds(base + (_NCHUNK - 4) * _C, _C)], osem0
    ).wait()
    pltpu.make_async_copy(
        rows1, out_hbm.at[pl.ds(base + (_NCHUNK - 3) * _C, _C)], osem1
    ).wait()
    pltpu.make_async_copy(
        rows2, out_hbm.at[pl.ds(base + (_NCHUNK - 2) * _C, _C)], osem2
    ).wait()
    pltpu.make_async_copy(
        rows3, out_hbm.at[pl.ds(base + (_NCHUNK - 1) * _C, _C)], osem3
    ).wait()


def kernel(x, table):
    xf = jnp.transpose(x.astype(jnp.int32)).reshape(_BN)
    out = _emb_lookup(xf, table)
    return jnp.swapaxes(out.reshape(_N, _B, _D), 0, 1)


# speedup vs baseline: 1.2160x; 1.2160x over previous
import functools

import jax
import jax.numpy as jnp
from jax import lax
from jax.experimental import pallas as pl
from jax.experimental.pallas import tpu as pltpu
from jax.experimental.pallas import tpu_sc as plsc

_B = 16384
_N = 26
_D = 128
_VOCAB_PER_FIELD = 1000
_LOG2_B = 14
_BN = _B * _N
_NW = 32
_BPW = _BN // _NW
_C = 256
_KIDX = _C // 128
_NCHUNK = _BPW // _C
_NBUF = 3

_mesh = plsc.VectorSubcoreMesh(core_axis_name="c", subcore_axis_name="s")


@functools.partial(
    pl.kernel,
    mesh=_mesh,
    out_type=jax.ShapeDtypeStruct((_BN, _D), jnp.float32),
    scratch_types=[
        pltpu.VMEM((_BPW,), jnp.int32),
        pltpu.VMEM((_BPW,), jnp.int32),
        pltpu.VMEM((_C, _D), jnp.float32),
        pltpu.VMEM((_C, _D), jnp.float32),
        pltpu.VMEM((_C, _D), jnp.float32),
        pltpu.SemaphoreType.DMA,
        pltpu.SemaphoreType.DMA,
        pltpu.SemaphoreType.DMA,
        pltpu.SemaphoreType.DMA,
    ],
)
def _emb_lookup(x_hbm, table_hbm, out_hbm,
                xv, idx_v, rows0, rows1, rows2,
                gsem, osem0, osem1, osem2):
    wid = lax.axis_index("s") * 2 + lax.axis_index("c")
    base = wid * _BPW

    pltpu.sync_copy(x_hbm.at[pl.ds(base, _BPW)], xv)

    def compute_idx(c):
        def add_body(i, _):
            sl = pl.ds(c * _C + i * 16, 16)
            pos = lax.iota(jnp.int32, 16) + (base + c * _C + i * 16)
            off = lax.shift_right_logical(pos, _LOG2_B) * _VOCAB_PER_FIELD
            idx_v[sl] = xv[sl] + off
            return ()

        lax.fori_loop(0, _C // 16, add_body, ())

    def chunk(c, rows, osem):
        @pl.when(c >= _NBUF)
        def _():
            pltpu.make_async_copy(
                rows, out_hbm.at[pl.ds(base + (c - _NBUF) * _C, _C)], osem
            ).wait()

        copies = [
            pltpu.async_copy(
                table_hbm.at[idx_v.at[pl.ds(c * _C + k * 128, 128)]],
                rows.at[pl.ds(k * 128, 128)],
                gsem,
            )
            for k in range(_KIDX)
        ]

        @pl.when(c + 1 < _NCHUNK)
        def _():
            compute_idx(c + 1)

        for cp in copies:
            cp.wait()
        pltpu.async_copy(rows, out_hbm.at[pl.ds(base + c * _C, _C)], osem)

    compute_idx(0)

    def body(i, _):
        chunk(_NBUF * i, rows0, osem0)
        chunk(_NBUF * i + 1, rows1, osem1)
        chunk(_NBUF * i + 2, rows2, osem2)
        return ()

    lax.fori_loop(0, _NCHUNK // _NBUF, body, ())
    chunk(_NCHUNK - 1, rows0, osem0)

    pltpu.make_async_copy(
        rows1, out_hbm.at[pl.ds(base + (_NCHUNK - 3) * _C, _C)], osem1
    ).wait()
    pltpu.make_async_copy(
        rows2, out_hbm.at[pl.ds(base + (_NCHUNK - 2) * _C, _C)], osem2
    ).wait()
    pltpu.make_async_copy(
        rows0, out_hbm.at[pl.ds(base + (_NCHUNK - 1) * _C, _C)], osem0
    ).wait()


def kernel(x, table):
    xf = jnp.transpose(x.astype(jnp.int32)).reshape(_BN)
    out = _emb_lookup(xf, table)
    return jnp.swapaxes(out.reshape(_N, _B, _D), 0, 1)
